# baseline (speedup 1.0000x reference)
import functools

import jax
import jax.numpy as jnp
from jax import lax
from jax.experimental import pallas as pl
from jax.experimental.pallas import tpu as pltpu
from jax.experimental.pallas import tpu_sc as plsc

A_PLUS = 0.04
A_MINUS = -0.04
INV_TAU = 100.0

L = 16
G = 64


@functools.lru_cache(maxsize=None)
def _build(M, H, B):
    info = plsc.get_sparse_core_info()
    NC, NS = info.num_cores, info.num_subcores
    NW = NC * NS
    assert H == L and M % NW == 0 and B % L == 0
    R = M // NW
    npart = 1
    while R % npart != 0 or (R // npart) * H > 50000:
        npart += 1
    CH = R // npart
    CHW = CH * H
    NG = B // L

    mesh = plsc.VectorSubcoreMesh(core_axis_name="c", subcore_axis_name="s")

    def body(mem_ref, dt_ref, idx_ref, out_ref,
             chunk, idxb, myev, pev, prow, dtb, sem_in, sem_g):
        wid = lax.axis_index("s") * NC + lax.axis_index("c")
        base = wid * R
        iota = lax.iota(jnp.int32, L)

        pltpu.sync_copy(idx_ref, idxb)

        def scan_body(g, off):
            iv = idxb[pl.ds(g * L, L)]
            m = (iv >= base) & (iv < base + R)
            pos = plsc.cumsum(m.astype(jnp.int32))
            plsc.store_scatter(myev, [off + pos - 1], g * L + iota, mask=m)
            return off + jnp.sum(m.astype(jnp.int32))

        n_my = lax.fori_loop(0, NG, scan_body, jnp.int32(0))
        ngm = (n_my + (L - 1)) >> 4

        def part_body(p, _):
            pbase = base + p * CH
            cp_in = pltpu.async_copy(
                mem_ref.at[pl.ds(pbase * H, CHW)], chunk, sem_in)

            def pscan(j, offp):
                valid = (j * L + iota) < n_my
                ev = myev[pl.ds(j * L, L)]
                gi = plsc.load_gather(idxb, [ev], mask=valid)
                pm = valid & (gi >= pbase) & (gi < pbase + CH)
                pos = offp + plsc.cumsum(pm.astype(jnp.int32)) - 1
                plsc.store_scatter(pev, [pos], ev, mask=pm)
                plsc.store_scatter(prow, [pos], gi - pbase, mask=pm)
                return offp + jnp.sum(pm.astype(jnp.int32))

            n_p = lax.fori_loop(0, ngm, pscan, jnp.int32(0))

            zz = jnp.zeros((L,), jnp.int32)

            def padb(k, _):
                pev[pl.ds(n_p + k * L, L)] = zz
                return 0

            lax.fori_loop(0, G // L, padb, 0)
            cp_in.wait()

            nch = (n_p + (G - 1)) >> 6

            def chunk_body(c, _):
                pltpu.async_copy(
                    dt_ref.at[pev.at[pl.ds(c * G, G)]], dtb, sem_g).wait()
                nj = jnp.minimum(n_p - c * G, G)

                def ev_body(j, _):
                    d = plsc.load_gather(
                        dtb, [jnp.full((L,), j, jnp.int32), iota])
                    pw = plsc.load_gather(
                        prow, [jnp.full((L,), c * G + j, jnp.int32)])
                    dw = jnp.where(
                        d > 0, A_PLUS * jnp.exp(d * (-INV_TAU)),
                        jnp.where(d < 0, A_MINUS * jnp.exp(d * INV_TAU),
                                  jnp.zeros_like(d)))
                    plsc.addupdate_scatter(chunk, [pw * H + iota], dw)
                    return 0

                lax.fori_loop(0, nj, ev_body, 0)
                return 0

            lax.fori_loop(0, nch, chunk_body, 0)
            pltpu.sync_copy(chunk, out_ref.at[pl.ds(pbase * H, CHW)])
            return 0

        lax.fori_loop(0, npart, part_body, 0)

    return pl.kernel(
        body,
        out_type=jax.ShapeDtypeStruct((M * H,), jnp.float32),
        mesh=mesh,
        compiler_params=pltpu.CompilerParams(
            needs_layout_passes=False, use_tc_tiling_on_sc=False),
        scratch_types=[
            pltpu.VMEM((CHW,), jnp.float32),
            pltpu.VMEM((B,), jnp.int32),
            pltpu.VMEM((B + L,), jnp.int32),
            pltpu.VMEM((B + G,), jnp.int32),
            pltpu.VMEM((B + L,), jnp.int32),
            pltpu.VMEM((G, L), jnp.float32),
            pltpu.SemaphoreType.DMA,
            pltpu.SemaphoreType.DMA,
        ],
    )


def kernel(mem, delta_t, idx):
    M, H = mem.shape
    B = idx.shape[0]
    out1 = _build(M, H, B)(
        mem.reshape(M * H), delta_t, idx.astype(jnp.int32))
    return out1.reshape(M, H)
